# Initial kernel scaffold; baseline (speedup 1.0000x reference)
import jax
import jax.numpy as jnp
from jax.experimental import pallas as pl


def kernel(x, edge_index, batch, ac, params):
    raise NotImplementedError("write your pallas kernel here")


# speedup vs baseline: 1.5816x; 1.5816x over previous
import functools

import jax
import jax.numpy as jnp
import numpy as np
from jax import lax
from jax.experimental import pallas as pl
from jax.experimental.pallas import tpu as pltpu
from jax.experimental.pallas import tpu_sc as plsc

N_NODES = 10000
N_EDGES = 320000
D = 128
G = 64
T = 5

NC = 2
NS = 16
NW = NC * NS
NPAD = 10240
RPW_NODES = NPAD // NS
CHUNK_ROWS = 4


@functools.cache
def _make_prop(n_chunks, weighted, with_counts):
    mesh = plsc.VectorSubcoreMesh(core_axis_name="c", subcore_axis_name="s",
                                  num_cores=NC, num_subcores=NS)
    rpw = n_chunks * CHUNK_ROWS

    out_types = [jax.ShapeDtypeStruct((NC, NPAD, D), jnp.float32)]
    if with_counts:
        out_types.append(jax.ShapeDtypeStruct((NC, NPAD), jnp.float32))
    scratch = [
        pltpu.VMEM((CHUNK_ROWS, 128), jnp.int32),
        pltpu.VMEM((CHUNK_ROWS, 128), jnp.int32),
        pltpu.VMEM((128, D), jnp.float32),
        pltpu.VMEM_SHARED((NPAD, D), jnp.float32),
        pltpu.SemaphoreType.DMA,
    ]
    if weighted:
        scratch.insert(2, pltpu.VMEM((CHUNK_ROWS * 128,), jnp.float32))
    if with_counts:
        scratch.append(pltpu.VMEM_SHARED((NPAD,), jnp.float32))
        scratch.append(pltpu.VMEM((128,), jnp.float32))
        scratch.append(pltpu.VMEM((RPW_NODES,), jnp.float32))

    def body(*refs):
        ins = refs[:3 + (1 if weighted else 0) + (2 if with_counts else 1)]
        pos = 0
        table_hbm, src_hbm, dst_hbm = ins[0], ins[1], ins[2]
        pos = 3
        w_hbm = ins[pos] if weighted else None
        pos += 1 if weighted else 0
        z2d_hbm = ins[pos]
        pos += 1
        z1d_hbm = ins[pos] if with_counts else None
        refs = refs[len(ins):]
        out_hbm = refs[0]
        deg_hbm = refs[1] if with_counts else None
        refs = refs[2 if with_counts else 1:]
        if weighted:
            sidx_v, didx_v, w_v, rows_v, acc_sh, sem = refs[:6]
            refs = refs[6:]
        else:
            sidx_v, didx_v, rows_v, acc_sh, sem = refs[:5]
            w_v = None
            refs = refs[5:]
        if with_counts:
            acc1_sh, ones_v, buf1_v = refs

        cid = lax.axis_index("c")
        sid = lax.axis_index("s")
        wid = sid * NC + cid
        nslc = pl.ds(sid * RPW_NODES, RPW_NODES)

        pltpu.sync_copy(z2d_hbm, rows_v)
        for q in range(RPW_NODES // 128):
            pltpu.sync_copy(
                rows_v, acc_sh.at[pl.ds(sid * RPW_NODES + q * 128, 128)])
        if with_counts:
            pltpu.sync_copy(z1d_hbm, buf1_v)
            pltpu.sync_copy(buf1_v, acc1_sh.at[nslc])
            for c in range(8):
                ones_v[pl.ds(c * 16, 16)] = jnp.ones((16,), jnp.float32)
        plsc.subcore_barrier()

        def chunk(k, carry):
            roff = wid * rpw + k * CHUNK_ROWS
            pltpu.sync_copy(src_hbm.at[pl.ds(roff, CHUNK_ROWS)], sidx_v)
            pltpu.sync_copy(dst_hbm.at[pl.ds(roff, CHUNK_ROWS)], didx_v)
            if weighted:
                pltpu.sync_copy(
                    w_hbm.at[pl.ds(roff * 128, CHUNK_ROWS * 128)], w_v)
            for j in range(CHUNK_ROWS):
                pltpu.async_copy(table_hbm.at[sidx_v.at[j]], rows_v,
                                 sem).wait()
                if weighted:
                    def scale_group(g, c2):
                        woff = pl.multiple_of(j * 128 + g * 16, 16)
                        wvec = w_v[pl.ds(woff, 16)]
                        for i in range(16):
                            ws = wvec[i]
                            row = g * 16 + i
                            for c in range(D // 16):
                                rows_v[row, pl.ds(c * 16, 16)] = (
                                    ws * rows_v[row, pl.ds(c * 16, 16)])
                        return c2
                    lax.fori_loop(0, 8, scale_group, 0)
                pltpu.sync_copy(rows_v, acc_sh.at[didx_v.at[j]], add=True)
                if with_counts:
                    pltpu.sync_copy(ones_v, acc1_sh.at[didx_v.at[j]],
                                    add=True)
            return carry
        lax.fori_loop(0, n_chunks, chunk, 0)
        plsc.subcore_barrier()

        for q in range(RPW_NODES // 128):
            qslc = pl.ds(sid * RPW_NODES + q * 128, 128)
            pltpu.sync_copy(acc_sh.at[qslc], rows_v)
            pltpu.sync_copy(rows_v, out_hbm.at[cid, qslc])
        if with_counts:
            pltpu.sync_copy(acc1_sh.at[nslc], buf1_v)
            pltpu.sync_copy(buf1_v, deg_hbm.at[cid, nslc])

    return pl.kernel(body, out_type=out_types, mesh=mesh,
                     scratch_types=scratch)


def _pad_table(table):
    return jnp.concatenate(
        [table, jnp.zeros((NPAD - table.shape[0], D), table.dtype)], axis=0)


def _sc_prop(table, src, dst, w=None, with_counts=False):
    e = src.shape[0]
    n_chunks = -(-e // (NW * 128 * CHUNK_ROWS))
    epad = n_chunks * NW * 128 * CHUNK_ROWS
    pad_idx = jnp.full((epad - e,), NPAD - 1, jnp.int32)
    src2 = jnp.concatenate([src, pad_idx]).reshape(-1, 128)
    dst2 = jnp.concatenate([dst, pad_idx]).reshape(-1, 128)
    args = [_pad_table(table) if table.shape[0] != NPAD else table,
            src2, dst2]
    if w is not None:
        args.append(jnp.concatenate(
            [w, jnp.zeros((epad - e,), jnp.float32)]))
    args.append(jnp.zeros((128, D), jnp.float32))
    if with_counts:
        args.append(jnp.zeros((RPW_NODES,), jnp.float32))
    fn = _make_prop(n_chunks, w is not None, with_counts)
    res = fn(*args)
    if isinstance(res, (list, tuple)):
        return tuple(res) if with_counts else res[0]
    return res


def _final_body(embT_ref, ac_ref, gw_ref, gb_ref, wk_ref, bk_ref, wq_ref,
                bq_ref, wo_ref, bo_ref, out_ref, dpp_ref):
    f32 = jnp.float32
    e = [embT_ref[pl.ds(t * G, G), :] for t in range(T)]

    def _bh(v):
        return v.astype(jnp.bfloat16).astype(f32)
    ne, neh = [], []
    for t in range(T):
        nrm = jnp.sqrt(jnp.sum(e[t] * e[t], axis=1, keepdims=True))
        nv = e[t] / jnp.maximum(nrm, 1e-12)
        ne.append(nv)
        neh.append(_bh(nv))
    a = [[None] * T for _ in range(T)]
    for t in range(T):
        for s in range(t + 1):
            g_ts = jnp.sum(neh[t] * neh[s], axis=1, keepdims=True)
            a[t][s] = g_ts
            a[s][t] = g_ts
    ld = jnp.zeros((G, 1), f32)
    for k in range(T):
        piv = a[k][k]
        ld = ld + jnp.log(piv)
        for i in range(k + 1, T):
            f = a[i][k] / piv
            for j in range(k + 1, T):
                a[i][j] = a[i][j] - f * a[k][j]
    dpp_ref[...] = ld

    gate = jax.lax.dot_general(
        ac_ref[...], gw_ref[...], (((1,), (0,)), ((), ())),
        preferred_element_type=f32) + gb_ref[...]

    dot = lambda x, w: jax.lax.dot_general(
        x, w, (((1,), (0,)), ((), ())), preferred_element_type=f32)
    q = [dot(e[t], wq_ref[...]) + bq_ref[...] for t in range(T)]
    k = [dot(e[t], wk_ref[...]) + bk_ref[...] for t in range(T)]
    qs = [_bh(v) for v in q]
    ks = [_bh(v) for v in k]
    inv_sqrt_d = 1.0 / np.sqrt(D)
    def _qk(t, s):
        return jnp.sum(qs[t] * ks[s], axis=1, keepdims=True)
    sc = [[_qk(t, s) * inv_sqrt_d * gate[:, s:s + 1] for s in range(T)]
          for t in range(T)]

    ctx_sum = jnp.zeros((G, D), f32)
    for t in range(T):
        row = sc[t]
        sel = []
        for s in range(T):
            cnt = jnp.zeros((G, 1), f32)
            for j in range(T):
                if j == s:
                    continue
                gtr = row[j] > row[s]
                if j < s:
                    gtr = gtr | (row[j] == row[s])
                cnt = cnt + gtr.astype(f32)
            sel.append(cnt < 3.0)
        m = jnp.full((G, 1), -1e30, f32)
        for s in range(T):
            m = jnp.maximum(m, jnp.where(sel[s], row[s], -1e30))
        ex = [jnp.where(sel[s], jnp.exp(row[s] - m), 0.0) for s in range(T)]
        den = ex[0]
        for s in range(1, T):
            den = den + ex[s]
        for s in range(T):
            ctx_sum = ctx_sum + (ex[s] / den) * e[s]

    out = dot(ctx_sum, wo_ref[...]) + float(T) * bo_ref[...]
    out_ref[...] = jnp.maximum(out, 0.0)


@jax.jit
def _final_stage(embT, ac, p):
    out, dpp = pl.pallas_call(
        _final_body,
        out_shape=[jax.ShapeDtypeStruct((G, D), jnp.float32),
                   jax.ShapeDtypeStruct((G, 1), jnp.float32)],
    )(embT, ac, p['gate_W'], p['gate_b'].reshape(1, T),
      p['att_Wk'], p['att_bk'].reshape(1, D),
      p['att_Wq'], p['att_bq'].reshape(1, D),
      p['att_Wo'], p['att_bo'].reshape(1, D))
    return out, dpp.reshape(G)


_BLK = 512
_NBLK = NPAD // _BLK


@functools.cache
def _make_mm(has_b, has_sb, has_so, has_bias, act):
    def body(*refs):
        i = 0
        a_ref = refs[i]; i += 1
        wa_ref = refs[i]; i += 1
        if has_b:
            b0_ref = refs[i]; b1_ref = refs[i + 1]; wb_ref = refs[i + 2]
            i += 3
        if has_sb:
            sb_ref = refs[i]; i += 1
        if has_so:
            so_ref = refs[i]; i += 1
        if has_bias:
            bias_ref = refs[i]; i += 1
        out_ref = refs[i]
        dot = lambda x, w: jax.lax.dot_general(
            x, w, (((1,), (0,)), ((), ())),
            preferred_element_type=jnp.float32)
        y = dot(a_ref[...], wa_ref[...])
        if has_b:
            b = b0_ref[...] + b1_ref[...]
            if has_sb:
                b = b * sb_ref[...]
            y = y + dot(b, wb_ref[...])
        if has_so:
            y = y * so_ref[...]
        if has_bias:
            y = y + bias_ref[...]
        if act:
            y = jnp.maximum(y, 0.0)
        out_ref[...] = y

    row = pl.BlockSpec((_BLK, D), lambda i: (i, 0))
    col = pl.BlockSpec((_BLK, 1), lambda i: (i, 0))
    full = pl.BlockSpec((D, D), lambda i: (0, 0))
    brow = pl.BlockSpec((1, D), lambda i: (0, 0))
    in_specs = [row, full]
    if has_b:
        in_specs += [row, row, full]
    if has_sb:
        in_specs += [col]
    if has_so:
        in_specs += [col]
    if has_bias:
        in_specs += [brow]
    return pl.pallas_call(
        body, grid=(_NBLK,), in_specs=in_specs, out_specs=row,
        out_shape=jax.ShapeDtypeStruct((NPAD, D), jnp.float32))


def _mm(a, wa, b=None, wb=None, sb=None, so=None, bias=None, act=False):
    args = [a, wa]
    if b is not None:
        args += [b[0], b[1], wb]
    if sb is not None:
        args.append(sb)
    if so is not None:
        args.append(so)
    if bias is not None:
        args.append(bias.reshape(1, D))
    fn = _make_mm(b is not None, sb is not None, so is not None,
                  bias is not None, act)
    return fn(*args)


@functools.cache
def _make_ew(has_c, has_s, act):
    def body(*refs):
        i = 0
        a_ref = refs[i]; b_ref = refs[i + 1]; i += 2
        if has_c:
            c_ref = refs[i]; i += 1
        if has_s:
            s_ref = refs[i]; i += 1
        bias_ref = refs[i]; out_ref = refs[i + 1]
        y = a_ref[...] + b_ref[...]
        if has_c:
            y = y + c_ref[...]
        if has_s:
            y = y * s_ref[...]
        y = y + bias_ref[...]
        if act:
            y = jnp.maximum(y, 0.0)
        out_ref[...] = y

    row = pl.BlockSpec((_BLK, D), lambda i: (i, 0))
    col = pl.BlockSpec((_BLK, 1), lambda i: (i, 0))
    brow = pl.BlockSpec((1, D), lambda i: (0, 0))
    in_specs = [row, row]
    if has_c:
        in_specs.append(row)
    if has_s:
        in_specs.append(col)
    in_specs.append(brow)
    return pl.pallas_call(
        body, grid=(_NBLK,), in_specs=in_specs, out_specs=row,
        out_shape=jax.ShapeDtypeStruct((NPAD, D), jnp.float32))


def _ew(a, b, c=None, s=None, bias=None, act=False):
    args = [a, b]
    if c is not None:
        args.append(c)
    if s is not None:
        args.append(s)
    args.append(bias.reshape(1, D))
    return _make_ew(c is not None, s is not None, act)(*args)


def _pool_body(b_ref, h1, h2, h3, h4, h5, out_ref, cnt_ref):
    i = pl.program_id(0)

    @pl.when(i == 0)
    def _init():
        out_ref[...] = jnp.zeros((T * G, D), jnp.float32)
        cnt_ref[...] = jnp.zeros((G, 1), jnp.float32)

    gids = jax.lax.broadcasted_iota(jnp.int32, (G, _BLK), 0)
    mask = (b_ref[0] == gids).astype(jnp.float32)
    cnt_ref[...] += jnp.sum(mask, axis=1, keepdims=True)
    dot = lambda m, h: jax.lax.dot_general(
        m, h, (((1,), (0,)), ((), ())), preferred_element_type=jnp.float32)
    for t, h in enumerate((h1, h2, h3, h4, h5)):
        out_ref[pl.ds(t * G, G), :] += dot(mask, h[...])

    @pl.when(i == _NBLK - 1)
    def _fin():
        c = jnp.maximum(cnt_ref[...], 1.0)
        for t in range(T):
            out_ref[pl.ds(t * G, G), :] = out_ref[pl.ds(t * G, G), :] / c


@functools.cache
def _make_pool():
    row = pl.BlockSpec((_BLK, D), lambda i: (i, 0))
    bspec = pl.BlockSpec((1, 1, _BLK), lambda i: (i, 0, 0))
    return pl.pallas_call(
        _pool_body, grid=(_NBLK,),
        in_specs=[bspec, row, row, row, row, row],
        out_specs=pl.BlockSpec((T * G, D), lambda i: (0, 0)),
        out_shape=jax.ShapeDtypeStruct((T * G, D), jnp.float32),
        scratch_shapes=[pltpu.VMEM((G, 1), jnp.float32)])


def _pool5(batchp, hs):
    b3 = batchp.reshape(_NBLK, 1, _BLK)
    return _make_pool()(b3, *hs)


def _seg_sum(vals, idx, n):
    return jax.ops.segment_sum(vals, idx, num_segments=n)


def _gcn_layer(xp, src, dst, dinv, W, b, act):
    u = _mm(xp, W, so=dinv)
    parts = _sc_prop(u, src, dst)
    return _ew(parts[0], parts[1], c=u, s=dinv, bias=b, act=act)


def _gat_layer(xp, srcA, dstA, W, Wsd, b, act):
    h = _mm(xp, W)
    hsd = _mm(h, Wsd)
    hs = hsd[:, 0]
    hd = hsd[:, 1]
    e = jax.nn.leaky_relu(hs[srcA] + hd[dstA], 0.2)
    m = jax.ops.segment_max(e, dstA, num_segments=N_NODES)
    ex = jnp.exp(e - m[dstA])
    denom = _seg_sum(ex, dstA, N_NODES)
    alpha = ex / jnp.maximum(denom[dstA], 1e-16)
    parts = _sc_prop(h, srcA, dstA, w=alpha)
    return _ew(parts[0], parts[1], bias=b, act=act)


def kernel(x, edge_index, batch, ac, params):
    p = params
    src = edge_index[0]
    dst = edge_index[1]
    si = jnp.arange(N_NODES, dtype=src.dtype)
    srcA = jnp.concatenate([src, si])
    dstA = jnp.concatenate([dst, si])
    xp = _pad_table(x)
    batchp = jnp.concatenate(
        [batch, jnp.full((NPAD - N_NODES,), G, batch.dtype)])

    px_parts, deg_parts = _sc_prop(xp, src, dst, with_counts=True)
    px0, px1 = px_parts[0], px_parts[1]
    deg = deg_parts[0] + deg_parts[1]
    dinv = jax.lax.rsqrt(deg + 1.0)[:, None]
    sinv = (1.0 / jnp.maximum(deg, 1.0))[:, None]

    h = _gcn_layer(xp, src, dst, dinv, p['gcn_W1'], p['gcn_b1'], True)
    gcn2 = _gcn_layer(h, src, dst, dinv, p['gcn_W2'], p['gcn_b2'], False)

    h = _mm(xp, p['sage_Wl1'], b=(px0, px1), wb=p['sage_Wr1'], sb=sinv,
            bias=p['sage_b1'], act=True)
    parts = _sc_prop(h, src, dst)
    sage2 = _mm(h, p['sage_Wl2'], b=parts, wb=p['sage_Wr2'], sb=sinv,
                bias=p['sage_b2'])

    wsd1 = jnp.zeros((D, D), jnp.float32).at[:, 0].set(
        p['gat_as1']).at[:, 1].set(p['gat_ad1'])
    wsd2 = jnp.zeros((D, D), jnp.float32).at[:, 0].set(
        p['gat_as2']).at[:, 1].set(p['gat_ad2'])
    h = _gat_layer(xp, srcA, dstA, p['gat_W1'], wsd1, p['gat_b1'], True)
    gat2 = _gat_layer(h, srcA, dstA, p['gat_W2'], wsd2, p['gat_b2'], False)

    h = _mm(xp, p['lin_W1'], bias=p['lin_b1'], act=True)
    lin2 = _mm(h, p['lin_W2'], bias=p['lin_b2'])

    h = _mm(xp, p['gin_W1'], b=(px0, px1), wb=p['gin_W1'],
            bias=p['gin_b1'], act=True)
    parts = _sc_prop(h, src, dst)
    gin2 = _mm(h, p['gin_W2'], b=parts, wb=p['gin_W2'], bias=p['gin_b2'])

    embT = _pool5(batchp, (gcn2, sage2, gat2, lin2, gin2))
    return _final_stage(embT, ac, params)
